# Initial kernel scaffold; baseline (speedup 1.0000x reference)
import jax
import jax.numpy as jnp
from jax.experimental import pallas as pl


def kernel(slice_matrices, qubit_embeddings, W1, b1, W2, b2):
    raise NotImplementedError("write your pallas kernel here")


# speedup vs baseline: 14.2918x; 14.2918x over previous
import functools

import jax
import jax.numpy as jnp
from jax import lax
from jax.experimental import pallas as pl
from jax.experimental.pallas import tpu as pltpu
from jax.experimental.pallas import tpu_sc as plsc

N = 10000
NPAD = 10240
D = 128
CW = 128
NTILES = 16
RPT = NPAD // NTILES
RB = 512


def _build_indices(slice_matrices, nb, e):
    per = e // NTILES
    nch = (per + CW - 1) // CW
    npad = nch * CW - per
    src = slice_matrices[:, 0, :].reshape(nb, NTILES, per)
    dst = slice_matrices[:, 1, :].reshape(nb, NTILES, per)
    if npad:
        lanes = (jnp.arange(npad, dtype=jnp.int32) * 7) % (NPAD - N)
        tspread = (jnp.arange(NTILES, dtype=jnp.int32) * 13)[:, None] % (NPAD - N)
        pad_src = N + (lanes[None, :] + tspread) % (NPAD - N)
        pad_dst = N + (lanes[None, :] + tspread + 97) % (NPAD - N)
        src = jnp.concatenate(
            [src, jnp.broadcast_to(pad_src[None], (nb, NTILES, npad))], axis=2)
        dst = jnp.concatenate(
            [dst, jnp.broadcast_to(pad_dst[None], (nb, NTILES, npad))], axis=2)
    src = src + (jnp.arange(nb, dtype=jnp.int32) * NPAD)[:, None, None]
    src_idx = src.reshape(nb, NTILES, nch, CW).astype(jnp.int32)
    dst_idx = dst.reshape(nb, NTILES, nch, CW).astype(jnp.int32)
    return src_idx, dst_idx, nch


def _deg_kernel(dst_idx, nb, nch):
    spc = nb // 2
    mesh = plsc.VectorSubcoreMesh(core_axis_name="c", subcore_axis_name="s")

    @functools.partial(
        pl.kernel, mesh=mesh,
        out_type=jax.ShapeDtypeStruct((nb, NPAD), jnp.float32),
        scratch_types=[
            pltpu.VMEM((nch, CW), jnp.int32),
            pltpu.VMEM((RPT,), jnp.float32),
            pltpu.VMEM_SHARED((NPAD,), jnp.float32),
        ],
    )
    def k(dst_hbm, deg_hbm, idx_v, ones_v, deg_sh):
        c = lax.axis_index("c")
        sid = lax.axis_index("s")
        for i in range(RPT // 16):
            ones_v[pl.ds(i * 16, 16)] = jnp.ones((16,), jnp.float32)
        r0 = sid * RPT
        for j in range(spc):
            s = c * spc + j
            pltpu.sync_copy(ones_v, deg_sh.at[pl.ds(r0, RPT)])
            pltpu.sync_copy(dst_hbm.at[s, sid], idx_v)
            plsc.subcore_barrier()

            def body(t, carry):
                pltpu.sync_copy(ones_v.at[pl.ds(0, CW)],
                                deg_sh.at[idx_v.at[t]], add=True)
                return carry

            lax.fori_loop(0, nch, body, 0)
            plsc.subcore_barrier()
            pltpu.sync_copy(deg_sh.at[pl.ds(r0, RPT)],
                            deg_hbm.at[s, pl.ds(r0, RPT)])

    return k(dst_idx)


def _aggregate_kernel(y, src_idx, dst_idx, nb, nch):
    spc = nb // 2
    mesh = plsc.VectorSubcoreMesh(core_axis_name="c", subcore_axis_name="s")

    @functools.partial(
        pl.kernel, mesh=mesh,
        out_type=jax.ShapeDtypeStruct((nb * NPAD, D), jnp.float32),
        scratch_types=[
            pltpu.VMEM((nch, CW), jnp.int32),
            pltpu.VMEM((nch, CW), jnp.int32),
            pltpu.VMEM((CW, D), jnp.float32),
            pltpu.VMEM_SHARED((NPAD, D), jnp.float32),
            pltpu.SemaphoreType.DMA,
        ],
    )
    def k(y_hbm, src_hbm, dst_hbm, out_hbm, src_v, dst_v, buf, acc_sh, sem):
        c = lax.axis_index("c")
        sid = lax.axis_index("s")
        r0 = sid * RPT
        for j in range(spc):
            s = c * spc + j
            base = s * NPAD
            pltpu.sync_copy(y_hbm.at[pl.ds(base + r0, RPT)],
                            acc_sh.at[pl.ds(r0, RPT)])
            pltpu.sync_copy(src_hbm.at[s, sid], src_v)
            pltpu.sync_copy(dst_hbm.at[s, sid], dst_v)
            plsc.subcore_barrier()

            def body(t, carry):
                pltpu.async_copy(y_hbm.at[src_v.at[t]], buf, sem).wait()
                pltpu.sync_copy(buf, acc_sh.at[dst_v.at[t]], add=True)
                return carry

            lax.fori_loop(0, nch, body, 0)
            plsc.subcore_barrier()
            pltpu.sync_copy(acc_sh.at[pl.ds(r0, RPT)],
                            out_hbm.at[pl.ds(base + r0, RPT)])

    return k(y, src_idx, dst_idx)


def _tc_first(qe_pad, deg4, w1, nb):
    def body(qe_ref, deg_ref, w_ref, y_ref):
        dinv = lax.rsqrt(deg_ref[0, 0, 0, :])
        xw = jnp.dot(qe_ref[...], w_ref[...],
                     preferred_element_type=jnp.float32,
                     precision=lax.Precision.HIGHEST)
        y_ref[0] = xw * dinv[:, None]

    return pl.pallas_call(
        body,
        grid=(nb, NPAD // RB),
        in_specs=[
            pl.BlockSpec((RB, D), lambda b, r: (r, 0)),
            pl.BlockSpec((1, 1, 1, RB), lambda b, r: (b, r, 0, 0)),
            pl.BlockSpec((D, D), lambda b, r: (0, 0)),
        ],
        out_specs=pl.BlockSpec((1, RB, D), lambda b, r: (b, r, 0)),
        out_shape=jax.ShapeDtypeStruct((nb, NPAD, D), jnp.float32),
    )(qe_pad, deg4, w1)


def _tc_mid(acc1, deg4, w2, b1, nb):
    def body(acc_ref, deg_ref, w_ref, b_ref, y_ref):
        dinv = lax.rsqrt(deg_ref[0, 0, 0, :])
        h = jnp.maximum(acc_ref[0] * dinv[:, None] + b_ref[0, :][None, :], 0.0)
        y = jnp.dot(h, w_ref[...],
                    preferred_element_type=jnp.float32,
                    precision=lax.Precision.HIGHEST)
        y_ref[0] = y * dinv[:, None]

    return pl.pallas_call(
        body,
        grid=(nb, NPAD // RB),
        in_specs=[
            pl.BlockSpec((1, RB, D), lambda b, r: (b, r, 0)),
            pl.BlockSpec((1, 1, 1, RB), lambda b, r: (b, r, 0, 0)),
            pl.BlockSpec((D, D), lambda b, r: (0, 0)),
            pl.BlockSpec((1, D), lambda b, r: (0, 0)),
        ],
        out_specs=pl.BlockSpec((1, RB, D), lambda b, r: (b, r, 0)),
        out_shape=jax.ShapeDtypeStruct((nb, NPAD, D), jnp.float32),
    )(acc1, deg4, w2, b1)


def _tc_last(acc2, deg4, b2, nb):
    def body(acc_ref, deg_ref, b_ref, o_ref):
        dinv = lax.rsqrt(deg_ref[0, 0, 0, :])
        o_ref[0] = jnp.maximum(acc_ref[0] * dinv[:, None] + b_ref[0, :][None, :], 0.0)

    return pl.pallas_call(
        body,
        grid=(nb, NPAD // RB),
        in_specs=[
            pl.BlockSpec((1, RB, D), lambda b, r: (b, r, 0)),
            pl.BlockSpec((1, 1, 1, RB), lambda b, r: (b, r, 0, 0)),
            pl.BlockSpec((1, D), lambda b, r: (0, 0)),
        ],
        out_specs=pl.BlockSpec((1, RB, D), lambda b, r: (b, r, 0)),
        out_shape=jax.ShapeDtypeStruct((nb, NPAD, D), jnp.float32),
    )(acc2, deg4, b2)


def kernel(slice_matrices, qubit_embeddings, W1, b1, W2, b2):
    nb = slice_matrices.shape[0]
    e = slice_matrices.shape[2]
    n = qubit_embeddings.shape[0]

    src_idx, dst_idx, nch = _build_indices(slice_matrices, nb, e)
    qe_pad = jnp.pad(qubit_embeddings, ((0, NPAD - n), (0, 0)))
    b1r = b1.reshape(1, D)
    b2r = b2.reshape(1, D)

    deg = _deg_kernel(dst_idx, nb, nch)
    deg4 = deg.reshape(nb, NPAD // RB, 1, RB)
    y1 = _tc_first(qe_pad, deg4, W1, nb)
    acc1 = _aggregate_kernel(y1.reshape(nb * NPAD, D), src_idx, dst_idx, nb, nch)
    y2 = _tc_mid(acc1.reshape(nb, NPAD, D), deg4, W2, b1r, nb)
    acc2 = _aggregate_kernel(y2.reshape(nb * NPAD, D), src_idx, dst_idx, nb, nch)
    out = _tc_last(acc2.reshape(nb, NPAD, D), deg4, b2r, nb)
    return out[:, :n, :].reshape(nb * n, D)


# speedup vs baseline: 17.7356x; 1.2410x over previous
import functools

import jax
import jax.numpy as jnp
from jax import lax
from jax.experimental import pallas as pl
from jax.experimental.pallas import tpu as pltpu
from jax.experimental.pallas import tpu_sc as plsc

N = 10000
NPAD = 10240
D = 128
CW = 64
NTILES = 16
RPT = NPAD // NTILES
RB = 512


def _build_indices(slice_matrices, nb, e):
    per = e // NTILES
    nch = (per + CW - 1) // CW
    nch = ((nch + 3) // 4) * 4
    npad = nch * CW - per
    src = slice_matrices[:, 0, :].reshape(nb, NTILES, per)
    dst = slice_matrices[:, 1, :].reshape(nb, NTILES, per)
    if npad:
        lanes = (jnp.arange(npad, dtype=jnp.int32) * 7) % (NPAD - N)
        tspread = (jnp.arange(NTILES, dtype=jnp.int32) * 13)[:, None] % (NPAD - N)
        pad_src = N + (lanes[None, :] + tspread) % (NPAD - N)
        pad_dst = N + (lanes[None, :] + tspread + 97) % (NPAD - N)
        src = jnp.concatenate(
            [src, jnp.broadcast_to(pad_src[None], (nb, NTILES, npad))], axis=2)
        dst = jnp.concatenate(
            [dst, jnp.broadcast_to(pad_dst[None], (nb, NTILES, npad))], axis=2)
    src = src + (jnp.arange(nb, dtype=jnp.int32) * NPAD)[:, None, None]
    src_idx = src.reshape(nb, NTILES, nch, CW).astype(jnp.int32)
    dst_idx = dst.reshape(nb, NTILES, nch, CW).astype(jnp.int32)
    return src_idx, dst_idx, nch


def _deg_kernel(dst_idx, nb, nch):
    spc = nb // 2
    mesh = plsc.VectorSubcoreMesh(core_axis_name="c", subcore_axis_name="s")

    @functools.partial(
        pl.kernel, mesh=mesh,
        out_type=jax.ShapeDtypeStruct((nb, NPAD), jnp.float32),
        scratch_types=[
            pltpu.VMEM((nch, CW), jnp.int32),
            pltpu.VMEM((RPT,), jnp.float32),
            pltpu.VMEM_SHARED((NPAD,), jnp.float32),
        ],
    )
    def k(dst_hbm, deg_hbm, idx_v, ones_v, deg_sh):
        c = lax.axis_index("c")
        sid = lax.axis_index("s")
        for i in range(RPT // 16):
            ones_v[pl.ds(i * 16, 16)] = jnp.ones((16,), jnp.float32)
        r0 = sid * RPT
        for j in range(spc):
            s = c * spc + j
            pltpu.sync_copy(ones_v, deg_sh.at[pl.ds(r0, RPT)])
            pltpu.sync_copy(dst_hbm.at[s, sid], idx_v)
            plsc.subcore_barrier()

            def body(t, carry):
                pltpu.sync_copy(ones_v.at[pl.ds(0, CW)],
                                deg_sh.at[idx_v.at[t]], add=True)
                return carry

            lax.fori_loop(0, nch, body, 0)
            plsc.subcore_barrier()
            pltpu.sync_copy(deg_sh.at[pl.ds(r0, RPT)],
                            deg_hbm.at[s, pl.ds(r0, RPT)])

    return k(dst_idx)


def _aggregate_kernel(y, src_idx, dst_idx, nb, nch):
    spc = nb // 2
    mesh = plsc.VectorSubcoreMesh(core_axis_name="c", subcore_axis_name="s")

    nbuf = 4
    lead = 2
    assert nch % nbuf == 0

    @functools.partial(
        pl.kernel, mesh=mesh,
        out_type=jax.ShapeDtypeStruct((nb * NPAD, D), jnp.float32),
        scratch_types=[
            pltpu.VMEM((nch, CW), jnp.int32),
            pltpu.VMEM((nch, CW), jnp.int32),
            pltpu.VMEM_SHARED((NPAD, D), jnp.float32),
        ] + [pltpu.VMEM((CW, D), jnp.float32) for _ in range(nbuf)]
          + [pltpu.SemaphoreType.DMA for _ in range(2 * nbuf)],
    )
    def k(y_hbm, src_hbm, dst_hbm, out_hbm, src_v, dst_v, acc_sh, *rest):
        bufs = rest[:nbuf]
        gsem = rest[nbuf:2 * nbuf]
        ssem = rest[2 * nbuf:3 * nbuf]
        c = lax.axis_index("c")
        sid = lax.axis_index("s")
        r0 = sid * RPT

        def run_slice(j, carry):
            s = c * spc + j
            base = s * NPAD
            pltpu.sync_copy(y_hbm.at[pl.ds(base + r0, RPT)],
                            acc_sh.at[pl.ds(r0, RPT)])
            pltpu.sync_copy(src_hbm.at[s, sid], src_v)
            pltpu.sync_copy(dst_hbm.at[s, sid], dst_v)
            plsc.subcore_barrier()

            def steps(g, carry):
                for b in range(nbuf):
                    tg = g * nbuf + b
                    ts = tg - lead

                    @pl.when(jnp.logical_and(tg < nch, tg >= nbuf))
                    def _():
                        pltpu.make_async_copy(
                            bufs[b], acc_sh.at[dst_v.at[tg - nbuf]],
                            ssem[b]).wait()

                    @pl.when(tg < nch)
                    def _():
                        pltpu.async_copy(y_hbm.at[src_v.at[tg]], bufs[b],
                                         gsem[b])

                    bs = (b - lead) % nbuf
                    @pl.when(jnp.logical_and(ts >= 0, ts < nch))
                    def _():
                        pltpu.make_async_copy(
                            y_hbm.at[src_v.at[ts]], bufs[bs], gsem[bs]).wait()
                        pltpu.async_copy(bufs[bs], acc_sh.at[dst_v.at[ts]],
                                         ssem[bs], add=True)
                return carry

            nsteps = (nch + lead + nbuf - 1) // nbuf
            lax.fori_loop(0, nsteps, steps, 0)
            for b in range(nbuf):
                tl = nch - nbuf + b
                pltpu.make_async_copy(
                    bufs[b], acc_sh.at[dst_v.at[tl]], ssem[b]).wait()
            plsc.subcore_barrier()
            pltpu.sync_copy(acc_sh.at[pl.ds(r0, RPT)],
                            out_hbm.at[pl.ds(base + r0, RPT)])
            return carry

        lax.fori_loop(0, spc, run_slice, 0)

    return k(y, src_idx, dst_idx)


def _tc_first(qe_pad, deg4, w1, nb):
    def body(qe_ref, deg_ref, w_ref, y_ref):
        dinv = lax.rsqrt(deg_ref[0, 0, 0, :])
        xw = jnp.dot(qe_ref[...], w_ref[...],
                     preferred_element_type=jnp.float32,
                     precision=lax.Precision.HIGHEST)
        y_ref[0] = xw * dinv[:, None]

    return pl.pallas_call(
        body,
        grid=(nb, NPAD // RB),
        in_specs=[
            pl.BlockSpec((RB, D), lambda b, r: (r, 0)),
            pl.BlockSpec((1, 1, 1, RB), lambda b, r: (b, r, 0, 0)),
            pl.BlockSpec((D, D), lambda b, r: (0, 0)),
        ],
        out_specs=pl.BlockSpec((1, RB, D), lambda b, r: (b, r, 0)),
        out_shape=jax.ShapeDtypeStruct((nb, NPAD, D), jnp.float32),
    )(qe_pad, deg4, w1)


def _tc_mid(acc1, deg4, w2, b1, nb):
    def body(acc_ref, deg_ref, w_ref, b_ref, y_ref):
        dinv = lax.rsqrt(deg_ref[0, 0, 0, :])
        h = jnp.maximum(acc_ref[0] * dinv[:, None] + b_ref[0, :][None, :], 0.0)
        y = jnp.dot(h, w_ref[...],
                    preferred_element_type=jnp.float32,
                    precision=lax.Precision.HIGHEST)
        y_ref[0] = y * dinv[:, None]

    return pl.pallas_call(
        body,
        grid=(nb, NPAD // RB),
        in_specs=[
            pl.BlockSpec((1, RB, D), lambda b, r: (b, r, 0)),
            pl.BlockSpec((1, 1, 1, RB), lambda b, r: (b, r, 0, 0)),
            pl.BlockSpec((D, D), lambda b, r: (0, 0)),
            pl.BlockSpec((1, D), lambda b, r: (0, 0)),
        ],
        out_specs=pl.BlockSpec((1, RB, D), lambda b, r: (b, r, 0)),
        out_shape=jax.ShapeDtypeStruct((nb, NPAD, D), jnp.float32),
    )(acc1, deg4, w2, b1)


def _tc_last(acc2, deg4, b2, nb):
    def body(acc_ref, deg_ref, b_ref, o_ref):
        dinv = lax.rsqrt(deg_ref[0, 0, 0, :])
        o_ref[0] = jnp.maximum(acc_ref[0] * dinv[:, None] + b_ref[0, :][None, :], 0.0)

    return pl.pallas_call(
        body,
        grid=(nb, NPAD // RB),
        in_specs=[
            pl.BlockSpec((1, RB, D), lambda b, r: (b, r, 0)),
            pl.BlockSpec((1, 1, 1, RB), lambda b, r: (b, r, 0, 0)),
            pl.BlockSpec((1, D), lambda b, r: (0, 0)),
        ],
        out_specs=pl.BlockSpec((1, RB, D), lambda b, r: (b, r, 0)),
        out_shape=jax.ShapeDtypeStruct((nb, NPAD, D), jnp.float32),
    )(acc2, deg4, b2)


def kernel(slice_matrices, qubit_embeddings, W1, b1, W2, b2):
    nb = slice_matrices.shape[0]
    e = slice_matrices.shape[2]
    n = qubit_embeddings.shape[0]

    src_idx, dst_idx, nch = _build_indices(slice_matrices, nb, e)
    qe_pad = jnp.pad(qubit_embeddings, ((0, NPAD - n), (0, 0)))
    b1r = b1.reshape(1, D)
    b2r = b2.reshape(1, D)

    deg = _deg_kernel(dst_idx, nb, nch)
    deg4 = deg.reshape(nb, NPAD // RB, 1, RB)
    y1 = _tc_first(qe_pad, deg4, W1, nb)
    acc1 = _aggregate_kernel(y1.reshape(nb * NPAD, D), src_idx, dst_idx, nb, nch)
    y2 = _tc_mid(acc1.reshape(nb, NPAD, D), deg4, W2, b1r, nb)
    acc2 = _aggregate_kernel(y2.reshape(nb * NPAD, D), src_idx, dst_idx, nb, nch)
    out = _tc_last(acc2.reshape(nb, NPAD, D), deg4, b2r, nb)
    return out[:, :n, :].reshape(nb * n, D)


# speedup vs baseline: 18.5074x; 1.0435x over previous
import functools

import jax
import jax.numpy as jnp
from jax import lax
from jax.experimental import pallas as pl
from jax.experimental.pallas import tpu as pltpu
from jax.experimental.pallas import tpu_sc as plsc

N = 10000
NPAD = 10240
D = 128
CW = 128
NTILES = 16
RPT = NPAD // NTILES
RB = 512


def _build_indices(slice_matrices, nb, e):
    per = e // NTILES
    nch = (per + CW - 1) // CW
    nch = ((nch + 1) // 2) * 2
    npad = nch * CW - per
    src = slice_matrices[:, 0, :].reshape(nb, NTILES, per)
    dst = slice_matrices[:, 1, :].reshape(nb, NTILES, per)
    if npad:
        lanes = (jnp.arange(npad, dtype=jnp.int32) * 7) % (NPAD - N)
        tspread = (jnp.arange(NTILES, dtype=jnp.int32) * 13)[:, None] % (NPAD - N)
        pad_src = N + (lanes[None, :] + tspread) % (NPAD - N)
        pad_dst = N + (lanes[None, :] + tspread + 97) % (NPAD - N)
        src = jnp.concatenate(
            [src, jnp.broadcast_to(pad_src[None], (nb, NTILES, npad))], axis=2)
        dst = jnp.concatenate(
            [dst, jnp.broadcast_to(pad_dst[None], (nb, NTILES, npad))], axis=2)
    src = src + (jnp.arange(nb, dtype=jnp.int32) * NPAD)[:, None, None]
    src_idx = src.reshape(nb, NTILES, nch, CW).astype(jnp.int32)
    dst_idx = dst.reshape(nb, NTILES, nch, CW).astype(jnp.int32)
    return src_idx, dst_idx, nch


def _deg_kernel(dst_idx, nb, nch):
    spc = nb // 2
    mesh = plsc.VectorSubcoreMesh(core_axis_name="c", subcore_axis_name="s")

    @functools.partial(
        pl.kernel, mesh=mesh,
        out_type=jax.ShapeDtypeStruct((nb, NPAD), jnp.float32),
        scratch_types=[
            pltpu.VMEM((nch, CW), jnp.int32),
            pltpu.VMEM((RPT,), jnp.float32),
            pltpu.VMEM_SHARED((NPAD,), jnp.float32),
        ],
    )
    def k(dst_hbm, deg_hbm, idx_v, ones_v, deg_sh):
        c = lax.axis_index("c")
        sid = lax.axis_index("s")
        for i in range(RPT // 16):
            ones_v[pl.ds(i * 16, 16)] = jnp.ones((16,), jnp.float32)
        r0 = sid * RPT
        for j in range(spc):
            s = c * spc + j
            pltpu.sync_copy(ones_v, deg_sh.at[pl.ds(r0, RPT)])
            pltpu.sync_copy(dst_hbm.at[s, sid], idx_v)
            plsc.subcore_barrier()

            def body(t, carry):
                pltpu.sync_copy(ones_v.at[pl.ds(0, CW)],
                                deg_sh.at[idx_v.at[t]], add=True)
                return carry

            lax.fori_loop(0, nch, body, 0)
            plsc.subcore_barrier()
            pltpu.sync_copy(deg_sh.at[pl.ds(r0, RPT)],
                            deg_hbm.at[s, pl.ds(r0, RPT)])

    return k(dst_idx)


def _aggregate_kernel(y, src_idx, dst_idx, nb, nch):
    spc = nb // 2
    mesh = plsc.VectorSubcoreMesh(core_axis_name="c", subcore_axis_name="s")

    nbuf = 2
    lead = 1
    assert nch % nbuf == 0

    @functools.partial(
        pl.kernel, mesh=mesh,
        out_type=jax.ShapeDtypeStruct((nb * NPAD, D), jnp.float32),
        scratch_types=[
            pltpu.VMEM((nch, CW), jnp.int32),
            pltpu.VMEM((nch, CW), jnp.int32),
            pltpu.VMEM_SHARED((NPAD, D), jnp.float32),
        ] + [pltpu.VMEM((CW, D), jnp.float32) for _ in range(nbuf)]
          + [pltpu.SemaphoreType.DMA for _ in range(2 * nbuf)],
    )
    def k(y_hbm, src_hbm, dst_hbm, out_hbm, src_v, dst_v, acc_sh, *rest):
        bufs = rest[:nbuf]
        gsem = rest[nbuf:2 * nbuf]
        ssem = rest[2 * nbuf:3 * nbuf]
        c = lax.axis_index("c")
        sid = lax.axis_index("s")
        r0 = sid * RPT

        def run_slice(j, carry):
            s = c * spc + j
            base = s * NPAD
            pltpu.sync_copy(y_hbm.at[pl.ds(base + r0, RPT)],
                            acc_sh.at[pl.ds(r0, RPT)])
            pltpu.sync_copy(src_hbm.at[s, sid], src_v)
            pltpu.sync_copy(dst_hbm.at[s, sid], dst_v)
            plsc.subcore_barrier()

            def steps(g, carry):
                for b in range(nbuf):
                    tg = g * nbuf + b
                    ts = tg - lead

                    @pl.when(jnp.logical_and(tg < nch, tg >= nbuf))
                    def _():
                        pltpu.make_async_copy(
                            bufs[b], acc_sh.at[dst_v.at[tg - nbuf]],
                            ssem[b]).wait()

                    @pl.when(tg < nch)
                    def _():
                        pltpu.async_copy(y_hbm.at[src_v.at[tg]], bufs[b],
                                         gsem[b])

                    bs = (b - lead) % nbuf
                    @pl.when(jnp.logical_and(ts >= 0, ts < nch))
                    def _():
                        pltpu.make_async_copy(
                            y_hbm.at[src_v.at[ts]], bufs[bs], gsem[bs]).wait()
                        pltpu.async_copy(bufs[bs], acc_sh.at[dst_v.at[ts]],
                                         ssem[bs], add=True)
                return carry

            nsteps = (nch + lead + nbuf - 1) // nbuf
            lax.fori_loop(0, nsteps, steps, 0)
            for b in range(nbuf):
                tl = nch - nbuf + b
                pltpu.make_async_copy(
                    bufs[b], acc_sh.at[dst_v.at[tl]], ssem[b]).wait()
            plsc.subcore_barrier()
            pltpu.sync_copy(acc_sh.at[pl.ds(r0, RPT)],
                            out_hbm.at[pl.ds(base + r0, RPT)])
            return carry

        lax.fori_loop(0, spc, run_slice, 0)

    return k(y, src_idx, dst_idx)


def _tc_first(qe_pad, deg4, w1, nb):
    def body(qe_ref, deg_ref, w_ref, y_ref):
        dinv = lax.rsqrt(deg_ref[0, 0, 0, :])
        xw = jnp.dot(qe_ref[...], w_ref[...], preferred_element_type=jnp.float32)
        y_ref[0] = xw * dinv[:, None]

    return pl.pallas_call(
        body,
        grid=(nb, NPAD // RB),
        in_specs=[
            pl.BlockSpec((RB, D), lambda b, r: (r, 0)),
            pl.BlockSpec((1, 1, 1, RB), lambda b, r: (b, r, 0, 0)),
            pl.BlockSpec((D, D), lambda b, r: (0, 0)),
        ],
        out_specs=pl.BlockSpec((1, RB, D), lambda b, r: (b, r, 0)),
        out_shape=jax.ShapeDtypeStruct((nb, NPAD, D), jnp.float32),
    )(qe_pad, deg4, w1)


def _tc_mid(acc1, deg4, w2, b1, nb):
    def body(acc_ref, deg_ref, w_ref, b_ref, y_ref):
        dinv = lax.rsqrt(deg_ref[0, 0, 0, :])
        h = jnp.maximum(acc_ref[0] * dinv[:, None] + b_ref[0, :][None, :], 0.0)
        y = jnp.dot(h, w_ref[...], preferred_element_type=jnp.float32)
        y_ref[0] = y * dinv[:, None]

    return pl.pallas_call(
        body,
        grid=(nb, NPAD // RB),
        in_specs=[
            pl.BlockSpec((1, RB, D), lambda b, r: (b, r, 0)),
            pl.BlockSpec((1, 1, 1, RB), lambda b, r: (b, r, 0, 0)),
            pl.BlockSpec((D, D), lambda b, r: (0, 0)),
            pl.BlockSpec((1, D), lambda b, r: (0, 0)),
        ],
        out_specs=pl.BlockSpec((1, RB, D), lambda b, r: (b, r, 0)),
        out_shape=jax.ShapeDtypeStruct((nb, NPAD, D), jnp.float32),
    )(acc1, deg4, w2, b1)


def _tc_last(acc2, deg4, b2, nb):
    def body(acc_ref, deg_ref, b_ref, o_ref):
        dinv = lax.rsqrt(deg_ref[0, 0, 0, :])
        o_ref[0] = jnp.maximum(acc_ref[0] * dinv[:, None] + b_ref[0, :][None, :], 0.0)

    return pl.pallas_call(
        body,
        grid=(nb, NPAD // RB),
        in_specs=[
            pl.BlockSpec((1, RB, D), lambda b, r: (b, r, 0)),
            pl.BlockSpec((1, 1, 1, RB), lambda b, r: (b, r, 0, 0)),
            pl.BlockSpec((1, D), lambda b, r: (0, 0)),
        ],
        out_specs=pl.BlockSpec((1, RB, D), lambda b, r: (b, r, 0)),
        out_shape=jax.ShapeDtypeStruct((nb, NPAD, D), jnp.float32),
    )(acc2, deg4, b2)


def kernel(slice_matrices, qubit_embeddings, W1, b1, W2, b2):
    nb = slice_matrices.shape[0]
    e = slice_matrices.shape[2]
    n = qubit_embeddings.shape[0]

    src_idx, dst_idx, nch = _build_indices(slice_matrices, nb, e)
    qe_pad = jnp.pad(qubit_embeddings, ((0, NPAD - n), (0, 0)))
    b1r = b1.reshape(1, D)
    b2r = b2.reshape(1, D)

    deg = _deg_kernel(dst_idx, nb, nch)
    deg4 = deg.reshape(nb, NPAD // RB, 1, RB)
    y1 = _tc_first(qe_pad, deg4, W1, nb)
    acc1 = _aggregate_kernel(y1.reshape(nb * NPAD, D), src_idx, dst_idx, nb, nch)
    y2 = _tc_mid(acc1.reshape(nb, NPAD, D), deg4, W2, b1r, nb)
    acc2 = _aggregate_kernel(y2.reshape(nb * NPAD, D), src_idx, dst_idx, nb, nch)
    out = _tc_last(acc2.reshape(nb, NPAD, D), deg4, b2r, nb)
    return out[:, :n, :].reshape(nb * n, D)


# speedup vs baseline: 25.3262x; 1.3684x over previous
import functools

import jax
import jax.numpy as jnp
from jax import lax
from jax.experimental import pallas as pl
from jax.experimental.pallas import tpu as pltpu
from jax.experimental.pallas import tpu_sc as plsc

N = 10000
NPAD = 10240
D = 128
CW = 128
NTILES = 16
RPT = NPAD // NTILES
RB = 2048


def _build_indices(slice_matrices, nb, e):
    per = e // NTILES
    nch = (per + CW - 1) // CW
    nch = ((nch + 1) // 2) * 2
    npad = nch * CW - per
    src = slice_matrices[:, 0, :].reshape(nb, NTILES, per)
    dst = slice_matrices[:, 1, :].reshape(nb, NTILES, per)
    if npad:
        lanes = (jnp.arange(npad, dtype=jnp.int32) * 7) % (NPAD - N)
        tspread = (jnp.arange(NTILES, dtype=jnp.int32) * 13)[:, None] % (NPAD - N)
        pad_src = N + (lanes[None, :] + tspread) % (NPAD - N)
        pad_dst = N + (lanes[None, :] + tspread + 97) % (NPAD - N)
        src = jnp.concatenate(
            [src, jnp.broadcast_to(pad_src[None], (nb, NTILES, npad))], axis=2)
        dst = jnp.concatenate(
            [dst, jnp.broadcast_to(pad_dst[None], (nb, NTILES, npad))], axis=2)
    src = src + (jnp.arange(nb, dtype=jnp.int32) * NPAD)[:, None, None]
    src_idx = src.reshape(nb, NTILES, nch, CW).astype(jnp.int32)
    dst_idx = dst.reshape(nb, NTILES, nch, CW).astype(jnp.int32)
    return src_idx, dst_idx, nch


def _deg_kernel(dst_idx, nb, nch):
    spc = nb // 2
    mesh = plsc.VectorSubcoreMesh(core_axis_name="c", subcore_axis_name="s")

    @functools.partial(
        pl.kernel, mesh=mesh,
        out_type=jax.ShapeDtypeStruct((nb, NPAD), jnp.float32),
        scratch_types=[
            pltpu.VMEM((nch, CW), jnp.int32),
            pltpu.VMEM((RPT,), jnp.float32),
            pltpu.VMEM_SHARED((NPAD,), jnp.float32),
        ],
    )
    def k(dst_hbm, deg_hbm, idx_v, ones_v, deg_sh):
        c = lax.axis_index("c")
        sid = lax.axis_index("s")
        for i in range(RPT // 16):
            ones_v[pl.ds(i * 16, 16)] = jnp.ones((16,), jnp.float32)
        r0 = sid * RPT
        for j in range(spc):
            s = c * spc + j
            pltpu.sync_copy(ones_v, deg_sh.at[pl.ds(r0, RPT)])
            pltpu.sync_copy(dst_hbm.at[s, sid], idx_v)
            plsc.subcore_barrier()

            def body(t, carry):
                pltpu.sync_copy(ones_v.at[pl.ds(0, CW)],
                                deg_sh.at[idx_v.at[t]], add=True)
                return carry

            lax.fori_loop(0, nch, body, 0)
            plsc.subcore_barrier()
            pltpu.sync_copy(deg_sh.at[pl.ds(r0, RPT)],
                            deg_hbm.at[s, pl.ds(r0, RPT)])

    return k(dst_idx)


def _aggregate_kernel(y, src_idx, dst_idx, nb, nch):
    spc = nb // 2
    mesh = plsc.VectorSubcoreMesh(core_axis_name="c", subcore_axis_name="s")

    nbuf = 2
    lead = 1
    assert nch % nbuf == 0

    @functools.partial(
        pl.kernel, mesh=mesh,
        out_type=jax.ShapeDtypeStruct((nb * NPAD, D), jnp.float32),
        scratch_types=[
            pltpu.VMEM((nch, CW), jnp.int32),
            pltpu.VMEM((nch, CW), jnp.int32),
            pltpu.VMEM_SHARED((NPAD, D), jnp.float32),
        ] + [pltpu.VMEM((CW, D), jnp.float32) for _ in range(nbuf)]
          + [pltpu.SemaphoreType.DMA for _ in range(2 * nbuf)],
    )
    def k(y_hbm, src_hbm, dst_hbm, out_hbm, src_v, dst_v, acc_sh, *rest):
        bufs = rest[:nbuf]
        gsem = rest[nbuf:2 * nbuf]
        ssem = rest[2 * nbuf:3 * nbuf]
        c = lax.axis_index("c")
        sid = lax.axis_index("s")
        r0 = sid * RPT

        def run_slice(j, carry):
            s = c * spc + j
            base = s * NPAD
            pltpu.sync_copy(y_hbm.at[pl.ds(base + r0, RPT)],
                            acc_sh.at[pl.ds(r0, RPT)])
            pltpu.sync_copy(src_hbm.at[s, sid], src_v)
            pltpu.sync_copy(dst_hbm.at[s, sid], dst_v)
            plsc.subcore_barrier()

            def steps(g, carry):
                for b in range(nbuf):
                    tg = g * nbuf + b
                    ts = tg - lead

                    @pl.when(jnp.logical_and(tg < nch, tg >= nbuf))
                    def _():
                        pltpu.make_async_copy(
                            bufs[b], acc_sh.at[dst_v.at[tg - nbuf]],
                            ssem[b]).wait()

                    @pl.when(tg < nch)
                    def _():
                        pltpu.async_copy(y_hbm.at[src_v.at[tg]], bufs[b],
                                         gsem[b])

                    bs = (b - lead) % nbuf
                    @pl.when(jnp.logical_and(ts >= 0, ts < nch))
                    def _():
                        pltpu.make_async_copy(
                            y_hbm.at[src_v.at[ts]], bufs[bs], gsem[bs]).wait()
                        pltpu.async_copy(bufs[bs], acc_sh.at[dst_v.at[ts]],
                                         ssem[bs], add=True)
                return carry

            nsteps = (nch + lead + nbuf - 1) // nbuf
            lax.fori_loop(0, nsteps, steps, 0)
            for b in range(nbuf):
                tl = nch - nbuf + b
                pltpu.make_async_copy(
                    bufs[b], acc_sh.at[dst_v.at[tl]], ssem[b]).wait()
            plsc.subcore_barrier()
            pltpu.sync_copy(acc_sh.at[pl.ds(r0, RPT)],
                            out_hbm.at[pl.ds(base + r0, RPT)])
            return carry

        lax.fori_loop(0, spc, run_slice, 0)

    return k(y, src_idx, dst_idx)


def _tc_first(qe_pad, deg4, w1, nb):
    def body(qe_ref, deg_ref, w_ref, y_ref):
        dinv = lax.rsqrt(deg_ref[0, 0, 0, :])
        xw = jnp.dot(qe_ref[...], w_ref[...], preferred_element_type=jnp.float32)
        y_ref[0] = xw * dinv[:, None]

    return pl.pallas_call(
        body,
        grid=(nb, NPAD // RB),
        in_specs=[
            pl.BlockSpec((RB, D), lambda b, r: (r, 0)),
            pl.BlockSpec((1, 1, 1, RB), lambda b, r: (b, r, 0, 0)),
            pl.BlockSpec((D, D), lambda b, r: (0, 0)),
        ],
        out_specs=pl.BlockSpec((1, RB, D), lambda b, r: (b, r, 0)),
        out_shape=jax.ShapeDtypeStruct((nb, NPAD, D), jnp.float32),
    )(qe_pad, deg4, w1)


def _tc_mid(acc1, deg4, w2, b1, nb):
    def body(acc_ref, deg_ref, w_ref, b_ref, y_ref):
        dinv = lax.rsqrt(deg_ref[0, 0, 0, :])
        h = jnp.maximum(acc_ref[0] * dinv[:, None] + b_ref[0, :][None, :], 0.0)
        y = jnp.dot(h, w_ref[...], preferred_element_type=jnp.float32)
        y_ref[0] = y * dinv[:, None]

    return pl.pallas_call(
        body,
        grid=(nb, NPAD // RB),
        in_specs=[
            pl.BlockSpec((1, RB, D), lambda b, r: (b, r, 0)),
            pl.BlockSpec((1, 1, 1, RB), lambda b, r: (b, r, 0, 0)),
            pl.BlockSpec((D, D), lambda b, r: (0, 0)),
            pl.BlockSpec((1, D), lambda b, r: (0, 0)),
        ],
        out_specs=pl.BlockSpec((1, RB, D), lambda b, r: (b, r, 0)),
        out_shape=jax.ShapeDtypeStruct((nb, NPAD, D), jnp.float32),
    )(acc1, deg4, w2, b1)


def _tc_last(acc2, deg4c, b2, nb):
    rbc = 2000

    def body(acc_ref, deg_ref, b_ref, o_ref):
        dinv = lax.rsqrt(deg_ref[0, 0, 0, :])
        o_ref[0] = jnp.maximum(acc_ref[0] * dinv[:, None] + b_ref[0, :][None, :], 0.0)

    return pl.pallas_call(
        body,
        grid=(nb, N // rbc),
        in_specs=[
            pl.BlockSpec((1, rbc, D), lambda b, r: (b, r, 0)),
            pl.BlockSpec((1, 1, 1, rbc), lambda b, r: (b, r, 0, 0)),
            pl.BlockSpec((1, D), lambda b, r: (0, 0)),
        ],
        out_specs=pl.BlockSpec((1, rbc, D), lambda b, r: (b, r, 0)),
        out_shape=jax.ShapeDtypeStruct((nb, N, D), jnp.float32),
    )(acc2, deg4c, b2)


def kernel(slice_matrices, qubit_embeddings, W1, b1, W2, b2):
    nb = slice_matrices.shape[0]
    e = slice_matrices.shape[2]
    n = qubit_embeddings.shape[0]

    src_idx, dst_idx, nch = _build_indices(slice_matrices, nb, e)
    qe_pad = jnp.pad(qubit_embeddings, ((0, NPAD - n), (0, 0)))
    b1r = b1.reshape(1, D)
    b2r = b2.reshape(1, D)

    deg = _deg_kernel(dst_idx, nb, nch)
    deg4 = deg.reshape(nb, NPAD // RB, 1, RB)
    deg4c = deg[:, :n].reshape(nb, 5, 1, 2000)
    y1 = _tc_first(qe_pad, deg4, W1, nb)
    acc1 = _aggregate_kernel(y1.reshape(nb * NPAD, D), src_idx, dst_idx, nb, nch)
    y2 = _tc_mid(acc1.reshape(nb, NPAD, D), deg4, W2, b1r, nb)
    acc2 = _aggregate_kernel(y2.reshape(nb * NPAD, D), src_idx, dst_idx, nb, nch)
    out = _tc_last(acc2.reshape(nb, NPAD, D), deg4c, b2r, nb)
    return out.reshape(nb * n, D)


# speedup vs baseline: 26.9006x; 1.0622x over previous
import functools

import jax
import jax.numpy as jnp
from jax import lax
from jax.experimental import pallas as pl
from jax.experimental.pallas import tpu as pltpu
from jax.experimental.pallas import tpu_sc as plsc

N = 10000
NPAD = 10240
D = 128
CW = 128
NTILES = 16
RPT = NPAD // NTILES
RB = 2048


def _g1(nb):
    return max(2, (nb // 5) * 2)


def _build_indices(slice_matrices, nb, e):
    per = e // NTILES
    nch = (per + CW - 1) // CW
    nch = ((nch + 1) // 2) * 2
    npad = nch * CW - per
    src = slice_matrices[:, 0, :].reshape(nb, NTILES, per)
    dst = slice_matrices[:, 1, :].reshape(nb, NTILES, per)
    if npad:
        lanes = (jnp.arange(npad, dtype=jnp.int32) * 7) % (NPAD - N)
        tspread = (jnp.arange(NTILES, dtype=jnp.int32) * 13)[:, None] % (NPAD - N)
        pad_src = N + (lanes[None, :] + tspread) % (NPAD - N)
        pad_dst = N + (lanes[None, :] + tspread + 97) % (NPAD - N)
        src = jnp.concatenate(
            [src, jnp.broadcast_to(pad_src[None], (nb, NTILES, npad))], axis=2)
        dst = jnp.concatenate(
            [dst, jnp.broadcast_to(pad_dst[None], (nb, NTILES, npad))], axis=2)
    g1 = _g1(nb)
    local = jnp.concatenate([jnp.arange(g1, dtype=jnp.int32),
                             jnp.arange(nb - g1, dtype=jnp.int32)])
    src = src + (local * NPAD)[:, None, None]
    src_idx = src.reshape(nb, NTILES, nch, CW).astype(jnp.int32)
    dst_idx = dst.reshape(nb, NTILES, nch, CW).astype(jnp.int32)
    return src_idx, dst_idx, nch


def _deg_kernel(dst_idx, nb, nch):
    spc = nb // 2
    mesh = plsc.VectorSubcoreMesh(core_axis_name="c", subcore_axis_name="s")

    @functools.partial(
        pl.kernel, mesh=mesh,
        out_type=jax.ShapeDtypeStruct((nb, NPAD), jnp.float32),
        scratch_types=[
            pltpu.VMEM((nch, CW), jnp.int32),
            pltpu.VMEM((RPT,), jnp.float32),
            pltpu.VMEM_SHARED((NPAD,), jnp.float32),
        ],
    )
    def k(dst_hbm, deg_hbm, idx_v, ones_v, deg_sh):
        c = lax.axis_index("c")
        sid = lax.axis_index("s")
        for i in range(RPT // 16):
            ones_v[pl.ds(i * 16, 16)] = jnp.ones((16,), jnp.float32)
        r0 = sid * RPT
        for j in range(spc):
            s = c * spc + j
            pltpu.sync_copy(ones_v, deg_sh.at[pl.ds(r0, RPT)])
            pltpu.sync_copy(dst_hbm.at[s, sid], idx_v)
            plsc.subcore_barrier()

            def body(t, carry):
                pltpu.sync_copy(ones_v.at[pl.ds(0, CW)],
                                deg_sh.at[idx_v.at[t]], add=True)
                return carry

            lax.fori_loop(0, nch, body, 0)
            plsc.subcore_barrier()
            pltpu.sync_copy(deg_sh.at[pl.ds(r0, RPT)],
                            deg_hbm.at[s, pl.ds(r0, RPT)])

    return k(dst_idx)


def _aggregate_kernel(y, src_idx, dst_idx, nb, nch):
    spc = nb // 2
    mesh = plsc.VectorSubcoreMesh(core_axis_name="c", subcore_axis_name="s")

    nbuf = 2
    lead = 1
    assert nch % nbuf == 0

    @functools.partial(
        pl.kernel, mesh=mesh,
        out_type=jax.ShapeDtypeStruct((nb * NPAD, D), jnp.float32),
        scratch_types=[
            pltpu.VMEM((nch, CW), jnp.int32),
            pltpu.VMEM((nch, CW), jnp.int32),
            pltpu.VMEM_SHARED((NPAD, D), jnp.float32),
        ] + [pltpu.VMEM((CW, D), jnp.float32) for _ in range(nbuf)]
          + [pltpu.SemaphoreType.DMA for _ in range(2 * nbuf)],
    )
    def k(y_hbm, src_hbm, dst_hbm, out_hbm, src_v, dst_v, acc_sh, *rest):
        bufs = rest[:nbuf]
        gsem = rest[nbuf:2 * nbuf]
        ssem = rest[2 * nbuf:3 * nbuf]
        c = lax.axis_index("c")
        sid = lax.axis_index("s")
        r0 = sid * RPT

        def run_slice(j, carry):
            s = c * spc + j
            base = s * NPAD
            pltpu.sync_copy(y_hbm.at[pl.ds(base + r0, RPT)],
                            acc_sh.at[pl.ds(r0, RPT)])
            pltpu.sync_copy(src_hbm.at[s, sid], src_v)
            pltpu.sync_copy(dst_hbm.at[s, sid], dst_v)
            plsc.subcore_barrier()

            def steps(g, carry):
                for b in range(nbuf):
                    tg = g * nbuf + b
                    ts = tg - lead

                    @pl.when(jnp.logical_and(tg < nch, tg >= nbuf))
                    def _():
                        pltpu.make_async_copy(
                            bufs[b], acc_sh.at[dst_v.at[tg - nbuf]],
                            ssem[b]).wait()

                    @pl.when(tg < nch)
                    def _():
                        pltpu.async_copy(y_hbm.at[src_v.at[tg]], bufs[b],
                                         gsem[b])

                    bs = (b - lead) % nbuf
                    @pl.when(jnp.logical_and(ts >= 0, ts < nch))
                    def _():
                        pltpu.make_async_copy(
                            y_hbm.at[src_v.at[ts]], bufs[bs], gsem[bs]).wait()
                        pltpu.async_copy(bufs[bs], acc_sh.at[dst_v.at[ts]],
                                         ssem[bs], add=True)
                return carry

            nsteps = (nch + lead + nbuf - 1) // nbuf
            lax.fori_loop(0, nsteps, steps, 0)
            for b in range(nbuf):
                tl = nch - nbuf + b
                pltpu.make_async_copy(
                    bufs[b], acc_sh.at[dst_v.at[tl]], ssem[b]).wait()
            plsc.subcore_barrier()
            pltpu.sync_copy(acc_sh.at[pl.ds(r0, RPT)],
                            out_hbm.at[pl.ds(base + r0, RPT)])
            return carry

        lax.fori_loop(0, spc, run_slice, 0)

    return k(y, src_idx, dst_idx)


def _tc_first(qe_pad, deg4, w1, nb):
    def body(qe_ref, deg_ref, w_ref, y_ref):
        dinv = lax.rsqrt(deg_ref[0, 0, 0, :])
        xw = jnp.dot(qe_ref[...], w_ref[...], preferred_element_type=jnp.float32)
        y_ref[0] = xw * dinv[:, None]

    return pl.pallas_call(
        body,
        grid=(nb, NPAD // RB),
        in_specs=[
            pl.BlockSpec((RB, D), lambda b, r: (r, 0)),
            pl.BlockSpec((1, 1, 1, RB), lambda b, r: (b, r, 0, 0)),
            pl.BlockSpec((D, D), lambda b, r: (0, 0)),
        ],
        out_specs=pl.BlockSpec((1, RB, D), lambda b, r: (b, r, 0)),
        out_shape=jax.ShapeDtypeStruct((nb, NPAD, D), jnp.float32),
    )(qe_pad, deg4, w1)


def _tc_mid(acc1, deg4, w2, b1, nb):
    def body(acc_ref, deg_ref, w_ref, b_ref, y_ref):
        dinv = lax.rsqrt(deg_ref[0, 0, 0, :])
        h = jnp.maximum(acc_ref[0] * dinv[:, None] + b_ref[0, :][None, :], 0.0)
        y = jnp.dot(h, w_ref[...], preferred_element_type=jnp.float32)
        y_ref[0] = y * dinv[:, None]

    return pl.pallas_call(
        body,
        grid=(nb, NPAD // RB),
        in_specs=[
            pl.BlockSpec((1, RB, D), lambda b, r: (b, r, 0)),
            pl.BlockSpec((1, 1, 1, RB), lambda b, r: (b, r, 0, 0)),
            pl.BlockSpec((D, D), lambda b, r: (0, 0)),
            pl.BlockSpec((1, D), lambda b, r: (0, 0)),
        ],
        out_specs=pl.BlockSpec((1, RB, D), lambda b, r: (b, r, 0)),
        out_shape=jax.ShapeDtypeStruct((nb, NPAD, D), jnp.float32),
    )(acc1, deg4, w2, b1)


def _tc_last(acc2, deg4c, b2, nb):
    rbc = 2000

    def body(acc_ref, deg_ref, b_ref, o_ref):
        dinv = lax.rsqrt(deg_ref[0, 0, 0, :])
        o_ref[0] = jnp.maximum(acc_ref[0] * dinv[:, None] + b_ref[0, :][None, :], 0.0)

    return pl.pallas_call(
        body,
        grid=(nb, N // rbc),
        in_specs=[
            pl.BlockSpec((1, rbc, D), lambda b, r: (b, r, 0)),
            pl.BlockSpec((1, 1, 1, rbc), lambda b, r: (b, r, 0, 0)),
            pl.BlockSpec((1, D), lambda b, r: (0, 0)),
        ],
        out_specs=pl.BlockSpec((1, rbc, D), lambda b, r: (b, r, 0)),
        out_shape=jax.ShapeDtypeStruct((nb, N, D), jnp.float32),
    )(acc2, deg4c, b2)


def kernel(slice_matrices, qubit_embeddings, W1, b1, W2, b2):
    nb = slice_matrices.shape[0]
    e = slice_matrices.shape[2]
    n = qubit_embeddings.shape[0]

    src_idx, dst_idx, nch = _build_indices(slice_matrices, nb, e)
    qe_pad = jnp.pad(qubit_embeddings, ((0, NPAD - n), (0, 0)))
    b1r = b1.reshape(1, D)
    b2r = b2.reshape(1, D)

    deg = _deg_kernel(dst_idx, nb, nch)
    deg4 = deg.reshape(nb, NPAD // RB, 1, RB)
    deg4c = deg[:, :n].reshape(nb, 5, 1, 2000)
    g1 = _g1(nb)

    def half(sl, g_nb):
        y1 = _tc_first(qe_pad, deg4[sl], W1, g_nb)
        acc1 = _aggregate_kernel(y1.reshape(g_nb * NPAD, D),
                                 src_idx[sl], dst_idx[sl], g_nb, nch)
        y2 = _tc_mid(acc1.reshape(g_nb, NPAD, D), deg4[sl], W2, b1r, g_nb)
        acc2 = _aggregate_kernel(y2.reshape(g_nb * NPAD, D),
                                 src_idx[sl], dst_idx[sl], g_nb, nch)
        return _tc_last(acc2.reshape(g_nb, NPAD, D), deg4c[sl], b2r, g_nb)

    out_a = half(slice(0, g1), g1)
    out_b = half(slice(g1, nb), nb - g1)
    return jnp.concatenate([out_a, out_b], axis=0).reshape(nb * n, D)


# speedup vs baseline: 28.6677x; 1.0657x over previous
import functools

import jax
import jax.numpy as jnp
from jax import lax
from jax.experimental import pallas as pl
from jax.experimental.pallas import tpu as pltpu
from jax.experimental.pallas import tpu_sc as plsc

N = 10000
NPAD = 10240
D = 128
CW = 125
NTILES = 16
RPT = NPAD // NTILES
RB = 2048


def _g1(nb):
    return max(2, (nb // 5) * 2)


def _build_indices(slice_matrices, nb, e):
    per = e // NTILES
    nch = (per + CW - 1) // CW
    nch = ((nch + 1) // 2) * 2
    npad = nch * CW - per
    src = slice_matrices[:, 0, :].reshape(nb, NTILES, per)
    dst = slice_matrices[:, 1, :].reshape(nb, NTILES, per)
    if npad:
        lanes = (jnp.arange(npad, dtype=jnp.int32) * 7) % (NPAD - N)
        tspread = (jnp.arange(NTILES, dtype=jnp.int32) * 13)[:, None] % (NPAD - N)
        pad_src = N + (lanes[None, :] + tspread) % (NPAD - N)
        pad_dst = N + (lanes[None, :] + tspread + 97) % (NPAD - N)
        src = jnp.concatenate(
            [src, jnp.broadcast_to(pad_src[None], (nb, NTILES, npad))], axis=2)
        dst = jnp.concatenate(
            [dst, jnp.broadcast_to(pad_dst[None], (nb, NTILES, npad))], axis=2)
    g1 = _g1(nb)
    local = jnp.concatenate([jnp.arange(g1, dtype=jnp.int32),
                             jnp.arange(nb - g1, dtype=jnp.int32)])
    src = src + (local * NPAD)[:, None, None]
    src_idx = src.reshape(nb, NTILES, nch, CW).astype(jnp.int32)
    dst_idx = dst.reshape(nb, NTILES, nch, CW).astype(jnp.int32)
    return src_idx, dst_idx, nch


def _deg_kernel(dst_idx, nb, nch):
    spc = nb // 2
    mesh = plsc.VectorSubcoreMesh(core_axis_name="c", subcore_axis_name="s")

    @functools.partial(
        pl.kernel, mesh=mesh,
        out_type=jax.ShapeDtypeStruct((nb, NPAD), jnp.float32),
        scratch_types=[
            pltpu.VMEM((nch, CW), jnp.int32),
            pltpu.VMEM((RPT,), jnp.float32),
            pltpu.VMEM_SHARED((NPAD,), jnp.float32),
        ],
    )
    def k(dst_hbm, deg_hbm, idx_v, ones_v, deg_sh):
        c = lax.axis_index("c")
        sid = lax.axis_index("s")
        for i in range(RPT // 16):
            ones_v[pl.ds(i * 16, 16)] = jnp.ones((16,), jnp.float32)
        r0 = sid * RPT
        for j in range(spc):
            s = c * spc + j
            pltpu.sync_copy(ones_v, deg_sh.at[pl.ds(r0, RPT)])
            pltpu.sync_copy(dst_hbm.at[s, sid], idx_v)
            plsc.subcore_barrier()

            def body(t, carry):
                pltpu.sync_copy(ones_v.at[pl.ds(0, CW)],
                                deg_sh.at[idx_v.at[t]], add=True)
                return carry

            lax.fori_loop(0, nch, body, 0)
            plsc.subcore_barrier()
            pltpu.sync_copy(deg_sh.at[pl.ds(r0, RPT)],
                            deg_hbm.at[s, pl.ds(r0, RPT)])

    return k(dst_idx)


def _aggregate_kernel(y, src_idx, dst_idx, nb, nch):
    spc = nb // 2
    mesh = plsc.VectorSubcoreMesh(core_axis_name="c", subcore_axis_name="s")

    nbuf = 2
    lead = 1
    assert nch % nbuf == 0

    @functools.partial(
        pl.kernel, mesh=mesh,
        out_type=jax.ShapeDtypeStruct((nb * NPAD, D), jnp.float32),
        scratch_types=[
            pltpu.VMEM((nch, CW), jnp.int32),
            pltpu.VMEM((nch, CW), jnp.int32),
            pltpu.VMEM_SHARED((NPAD, D), jnp.float32),
        ] + [pltpu.VMEM((CW, D), jnp.float32) for _ in range(nbuf)]
          + [pltpu.SemaphoreType.DMA for _ in range(2 * nbuf)],
    )
    def k(y_hbm, src_hbm, dst_hbm, out_hbm, src_v, dst_v, acc_sh, *rest):
        bufs = rest[:nbuf]
        gsem = rest[nbuf:2 * nbuf]
        ssem = rest[2 * nbuf:3 * nbuf]
        c = lax.axis_index("c")
        sid = lax.axis_index("s")
        r0 = sid * RPT

        def run_slice(j, carry):
            s = c * spc + j
            base = s * NPAD
            pltpu.sync_copy(y_hbm.at[pl.ds(base + r0, RPT)],
                            acc_sh.at[pl.ds(r0, RPT)])
            pltpu.sync_copy(src_hbm.at[s, sid], src_v)
            pltpu.sync_copy(dst_hbm.at[s, sid], dst_v)
            plsc.subcore_barrier()

            def steps(g, carry):
                for b in range(nbuf):
                    tg = g * nbuf + b
                    ts = tg - lead

                    @pl.when(jnp.logical_and(tg < nch, tg >= nbuf))
                    def _():
                        pltpu.make_async_copy(
                            bufs[b], acc_sh.at[dst_v.at[tg - nbuf]],
                            ssem[b]).wait()

                    @pl.when(tg < nch)
                    def _():
                        pltpu.async_copy(y_hbm.at[src_v.at[tg]], bufs[b],
                                         gsem[b])

                    bs = (b - lead) % nbuf
                    @pl.when(jnp.logical_and(ts >= 0, ts < nch))
                    def _():
                        pltpu.make_async_copy(
                            y_hbm.at[src_v.at[ts]], bufs[bs], gsem[bs]).wait()
                        pltpu.async_copy(bufs[bs], acc_sh.at[dst_v.at[ts]],
                                         ssem[bs], add=True)
                return carry

            nsteps = (nch + lead + nbuf - 1) // nbuf
            lax.fori_loop(0, nsteps, steps, 0)
            for b in range(nbuf):
                tl = nch - nbuf + b
                pltpu.make_async_copy(
                    bufs[b], acc_sh.at[dst_v.at[tl]], ssem[b]).wait()
            plsc.subcore_barrier()
            pltpu.sync_copy(acc_sh.at[pl.ds(r0, RPT)],
                            out_hbm.at[pl.ds(base + r0, RPT)])
            return carry

        lax.fori_loop(0, spc, run_slice, 0)

    return k(y, src_idx, dst_idx)


def _tc_first(qe_pad, deg4, w1, nb):
    def body(qe_ref, deg_ref, w_ref, y_ref):
        dinv = lax.rsqrt(deg_ref[0, 0, 0, :])
        xw = jnp.dot(qe_ref[...], w_ref[...], preferred_element_type=jnp.float32)
        y_ref[0] = xw * dinv[:, None]

    return pl.pallas_call(
        body,
        grid=(nb, NPAD // RB),
        in_specs=[
            pl.BlockSpec((RB, D), lambda b, r: (r, 0)),
            pl.BlockSpec((1, 1, 1, RB), lambda b, r: (b, r, 0, 0)),
            pl.BlockSpec((D, D), lambda b, r: (0, 0)),
        ],
        out_specs=pl.BlockSpec((1, RB, D), lambda b, r: (b, r, 0)),
        out_shape=jax.ShapeDtypeStruct((nb, NPAD, D), jnp.float32),
    )(qe_pad, deg4, w1)


def _tc_mid(acc1, deg4, w2, b1, nb):
    def body(acc_ref, deg_ref, w_ref, b_ref, y_ref):
        dinv = lax.rsqrt(deg_ref[0, 0, 0, :])
        h = jnp.maximum(acc_ref[0] * dinv[:, None] + b_ref[0, :][None, :], 0.0)
        y = jnp.dot(h, w_ref[...], preferred_element_type=jnp.float32)
        y_ref[0] = y * dinv[:, None]

    return pl.pallas_call(
        body,
        grid=(nb, NPAD // RB),
        in_specs=[
            pl.BlockSpec((1, RB, D), lambda b, r: (b, r, 0)),
            pl.BlockSpec((1, 1, 1, RB), lambda b, r: (b, r, 0, 0)),
            pl.BlockSpec((D, D), lambda b, r: (0, 0)),
            pl.BlockSpec((1, D), lambda b, r: (0, 0)),
        ],
        out_specs=pl.BlockSpec((1, RB, D), lambda b, r: (b, r, 0)),
        out_shape=jax.ShapeDtypeStruct((nb, NPAD, D), jnp.float32),
    )(acc1, deg4, w2, b1)


def _tc_last(acc2, deg4c, b2, g_nb, nb, base, prev=None):
    rbc = 2000

    def body(*refs):
        acc_ref, deg_ref, b_ref = refs[0], refs[1], refs[2]
        o_ref = refs[-1]
        dinv = lax.rsqrt(deg_ref[0, 0, 0, :])
        o_ref[0] = jnp.maximum(acc_ref[0] * dinv[:, None] + b_ref[0, :][None, :], 0.0)

    ins = [acc2, deg4c, b2]
    in_specs = [
        pl.BlockSpec((1, rbc, D), lambda b, r: (b, r, 0)),
        pl.BlockSpec((1, 1, 1, rbc), lambda b, r: (b, r, 0, 0)),
        pl.BlockSpec((1, D), lambda b, r: (0, 0)),
    ]
    aliases = {}
    if prev is not None:
        ins.append(prev)
        in_specs.append(pl.BlockSpec(memory_space=pl.ANY))
        aliases = {3: 0}
    return pl.pallas_call(
        body,
        grid=(g_nb, N // rbc),
        in_specs=in_specs,
        out_specs=pl.BlockSpec((1, rbc, D), lambda b, r: (b + base, r, 0)),
        out_shape=jax.ShapeDtypeStruct((nb, N, D), jnp.float32),
        input_output_aliases=aliases,
    )(*ins)


def kernel(slice_matrices, qubit_embeddings, W1, b1, W2, b2):
    nb = slice_matrices.shape[0]
    e = slice_matrices.shape[2]
    n = qubit_embeddings.shape[0]

    src_idx, dst_idx, nch = _build_indices(slice_matrices, nb, e)
    qe_pad = jnp.pad(qubit_embeddings, ((0, NPAD - n), (0, 0)))
    b1r = b1.reshape(1, D)
    b2r = b2.reshape(1, D)

    deg = _deg_kernel(dst_idx, nb, nch)
    deg4 = deg.reshape(nb, NPAD // RB, 1, RB)
    deg4c = deg[:, :n].reshape(nb, 5, 1, 2000)
    g1 = _g1(nb)

    def half(sl, g_nb, base, prev):
        y1 = _tc_first(qe_pad, deg4[sl], W1, g_nb)
        acc1 = _aggregate_kernel(y1.reshape(g_nb * NPAD, D),
                                 src_idx[sl], dst_idx[sl], g_nb, nch)
        y2 = _tc_mid(acc1.reshape(g_nb, NPAD, D), deg4[sl], W2, b1r, g_nb)
        acc2 = _aggregate_kernel(y2.reshape(g_nb * NPAD, D),
                                 src_idx[sl], dst_idx[sl], g_nb, nch)
        return _tc_last(acc2.reshape(g_nb, NPAD, D), deg4c[sl], b2r,
                        g_nb, nb, base, prev)

    out = half(slice(0, g1), g1, 0, None)
    out = half(slice(g1, nb), nb - g1, g1, out)
    return out.reshape(nb * n, D)


# speedup vs baseline: 28.8874x; 1.0077x over previous
import functools

import jax
import jax.numpy as jnp
from jax import lax
from jax.experimental import pallas as pl
from jax.experimental.pallas import tpu as pltpu
from jax.experimental.pallas import tpu_sc as plsc

N = 10000
NPAD = 10240
D = 128
CW = 125
NTILES = 16
RPT = NPAD // NTILES
RB = 2048


def _g1(nb):
    return max(2, (nb // 5) * 2)


def _build_indices(slice_matrices, nb, e):
    per = e // NTILES
    nch = (per + CW - 1) // CW
    nch = ((nch + 1) // 2) * 2
    npad = nch * CW - per
    src = slice_matrices[:, 0, :].reshape(nb, NTILES, per)
    dst = slice_matrices[:, 1, :].reshape(nb, NTILES, per)
    if npad:
        lanes = (jnp.arange(npad, dtype=jnp.int32) * 7) % (NPAD - N)
        tspread = (jnp.arange(NTILES, dtype=jnp.int32) * 13)[:, None] % (NPAD - N)
        pad_src = N + (lanes[None, :] + tspread) % (NPAD - N)
        pad_dst = N + (lanes[None, :] + tspread + 97) % (NPAD - N)
        src = jnp.concatenate(
            [src, jnp.broadcast_to(pad_src[None], (nb, NTILES, npad))], axis=2)
        dst = jnp.concatenate(
            [dst, jnp.broadcast_to(pad_dst[None], (nb, NTILES, npad))], axis=2)
    g1 = _g1(nb)
    local = jnp.concatenate([jnp.arange(g1, dtype=jnp.int32),
                             jnp.arange(nb - g1, dtype=jnp.int32)])
    src = src + (local * NPAD)[:, None, None]
    src_idx = src.reshape(nb, NTILES, nch, CW).astype(jnp.int32)
    dst_idx = dst.reshape(nb, NTILES, nch, CW).astype(jnp.int32)
    return src_idx, dst_idx, nch


def _deg_kernel(dst_idx, nb, nch):
    spc = nb // 2
    mesh = plsc.VectorSubcoreMesh(core_axis_name="c", subcore_axis_name="s")

    @functools.partial(
        pl.kernel, mesh=mesh,
        out_type=jax.ShapeDtypeStruct((nb, NPAD), jnp.float32),
        scratch_types=[
            pltpu.VMEM((nch, CW), jnp.int32),
            pltpu.VMEM((RPT,), jnp.float32),
            pltpu.VMEM_SHARED((NPAD,), jnp.float32),
        ],
    )
    def k(dst_hbm, deg_hbm, idx_v, ones_v, deg_sh):
        c = lax.axis_index("c")
        sid = lax.axis_index("s")
        for i in range(RPT // 16):
            ones_v[pl.ds(i * 16, 16)] = jnp.ones((16,), jnp.float32)
        r0 = sid * RPT
        for j in range(spc):
            s = c * spc + j
            pltpu.sync_copy(ones_v, deg_sh.at[pl.ds(r0, RPT)])
            pltpu.sync_copy(dst_hbm.at[s, sid], idx_v)
            plsc.subcore_barrier()

            def body(t, carry):
                pltpu.sync_copy(ones_v.at[pl.ds(0, CW)],
                                deg_sh.at[idx_v.at[t]], add=True)
                return carry

            lax.fori_loop(0, nch, body, 0)
            plsc.subcore_barrier()
            pltpu.sync_copy(deg_sh.at[pl.ds(r0, RPT)],
                            deg_hbm.at[s, pl.ds(r0, RPT)])

    return k(dst_idx)


def _aggregate_kernel(y, src_idx, dst_idx, nb, nch):
    spc = nb // 2
    mesh = plsc.VectorSubcoreMesh(core_axis_name="c", subcore_axis_name="s")

    nbuf = 2
    lead = 1
    assert nch % nbuf == 0

    @functools.partial(
        pl.kernel, mesh=mesh,
        out_type=jax.ShapeDtypeStruct((nb * NPAD, D), jnp.float32),
        scratch_types=[
            pltpu.VMEM((nch, CW), jnp.int32),
            pltpu.VMEM((nch, CW), jnp.int32),
            pltpu.VMEM_SHARED((NPAD, D), jnp.float32),
        ] + [pltpu.VMEM((CW, D), jnp.float32) for _ in range(nbuf)]
          + [pltpu.SemaphoreType.DMA for _ in range(2 * nbuf)],
    )
    def k(y_hbm, src_hbm, dst_hbm, out_hbm, src_v, dst_v, acc_sh, *rest):
        bufs = rest[:nbuf]
        gsem = rest[nbuf:2 * nbuf]
        ssem = rest[2 * nbuf:3 * nbuf]
        c = lax.axis_index("c")
        sid = lax.axis_index("s")
        r0 = sid * RPT

        def run_slice(j, carry):
            s = c * spc + j
            base = s * NPAD
            pltpu.sync_copy(y_hbm.at[pl.ds(base + r0, RPT)],
                            acc_sh.at[pl.ds(r0, RPT)])
            pltpu.sync_copy(src_hbm.at[s, sid], src_v)
            pltpu.sync_copy(dst_hbm.at[s, sid], dst_v)
            plsc.subcore_barrier()

            def steps(g, carry):
                for b in range(nbuf):
                    tg = g * nbuf + b
                    ts = tg - lead

                    @pl.when(jnp.logical_and(tg < nch, tg >= nbuf))
                    def _():
                        pltpu.make_async_copy(
                            bufs[b], acc_sh.at[dst_v.at[tg - nbuf]],
                            ssem[b]).wait()

                    @pl.when(tg < nch)
                    def _():
                        pltpu.async_copy(y_hbm.at[src_v.at[tg]], bufs[b],
                                         gsem[b])

                    bs = (b - lead) % nbuf
                    @pl.when(jnp.logical_and(ts >= 0, ts < nch))
                    def _():
                        pltpu.make_async_copy(
                            y_hbm.at[src_v.at[ts]], bufs[bs], gsem[bs]).wait()
                        pltpu.async_copy(bufs[bs], acc_sh.at[dst_v.at[ts]],
                                         ssem[bs], add=True)
                return carry

            nsteps = (nch + lead + nbuf - 1) // nbuf
            lax.fori_loop(0, nsteps, steps, 0)
            for b in range(nbuf):
                tl = nch - nbuf + b
                pltpu.make_async_copy(
                    bufs[b], acc_sh.at[dst_v.at[tl]], ssem[b]).wait()
            plsc.subcore_barrier()
            pltpu.sync_copy(acc_sh.at[pl.ds(r0, RPT)],
                            out_hbm.at[pl.ds(base + r0, RPT)])
            return carry

        lax.fori_loop(0, spc, run_slice, 0)

    return k(y, src_idx, dst_idx)


def _tc_first(qe_pad, deg4, w1, nb):
    def body(qe_ref, deg_ref, w_ref, y_ref):
        dinv = lax.rsqrt(deg_ref[0, 0, 0, :])
        xw = jnp.dot(qe_ref[...], w_ref[...], preferred_element_type=jnp.float32)
        y_ref[0] = xw * dinv[:, None]

    return pl.pallas_call(
        body,
        grid=(nb, NPAD // RB),
        in_specs=[
            pl.BlockSpec((RB, D), lambda b, r: (r, 0)),
            pl.BlockSpec((1, 1, 1, RB), lambda b, r: (b, r, 0, 0)),
            pl.BlockSpec((D, D), lambda b, r: (0, 0)),
        ],
        out_specs=pl.BlockSpec((1, RB, D), lambda b, r: (b, r, 0)),
        out_shape=jax.ShapeDtypeStruct((nb, NPAD, D), jnp.float32),
    )(qe_pad, deg4, w1)


def _tc_mid(acc1, deg4, w2, b1, nb):
    def body(acc_ref, deg_ref, w_ref, b_ref, y_ref):
        dinv = lax.rsqrt(deg_ref[0, 0, 0, :])
        h = jnp.maximum(acc_ref[0] * dinv[:, None] + b_ref[0, :][None, :], 0.0)
        y = jnp.dot(h, w_ref[...], preferred_element_type=jnp.float32)
        y_ref[0] = y * dinv[:, None]

    return pl.pallas_call(
        body,
        grid=(nb, NPAD // RB),
        in_specs=[
            pl.BlockSpec((1, RB, D), lambda b, r: (b, r, 0)),
            pl.BlockSpec((1, 1, 1, RB), lambda b, r: (b, r, 0, 0)),
            pl.BlockSpec((D, D), lambda b, r: (0, 0)),
            pl.BlockSpec((1, D), lambda b, r: (0, 0)),
        ],
        out_specs=pl.BlockSpec((1, RB, D), lambda b, r: (b, r, 0)),
        out_shape=jax.ShapeDtypeStruct((nb, NPAD, D), jnp.float32),
    )(acc1, deg4, w2, b1)


def _tc_last(acc2, deg4c, b2, g_nb, nb, base, prev=None):
    rbc = 2000

    def body(*refs):
        acc_ref, deg_ref, b_ref = refs[0], refs[1], refs[2]
        o_ref = refs[-1]
        dinv = lax.rsqrt(deg_ref[0, 0, 0, :])
        o_ref[0] = jnp.maximum(acc_ref[0] * dinv[:, None] + b_ref[0, :][None, :], 0.0)

    ins = [acc2, deg4c, b2]
    in_specs = [
        pl.BlockSpec((1, rbc, D), lambda b, r: (b, r, 0)),
        pl.BlockSpec((1, 1, 1, rbc), lambda b, r: (b, r, 0, 0)),
        pl.BlockSpec((1, D), lambda b, r: (0, 0)),
    ]
    aliases = {}
    if prev is not None:
        ins.append(prev)
        in_specs.append(pl.BlockSpec(memory_space=pl.ANY))
        aliases = {3: 0}
    return pl.pallas_call(
        body,
        grid=(g_nb, N // rbc),
        in_specs=in_specs,
        out_specs=pl.BlockSpec((1, rbc, D), lambda b, r: (b + base, r, 0)),
        out_shape=jax.ShapeDtypeStruct((nb, N, D), jnp.float32),
        input_output_aliases=aliases,
    )(*ins)


def kernel(slice_matrices, qubit_embeddings, W1, b1, W2, b2):
    nb = slice_matrices.shape[0]
    e = slice_matrices.shape[2]
    n = qubit_embeddings.shape[0]

    src_idx, dst_idx, nch = _build_indices(slice_matrices, nb, e)
    qe_pad = jnp.pad(qubit_embeddings, ((0, NPAD - n), (0, 0)))
    b1r = b1.reshape(1, D)
    b2r = b2.reshape(1, D)

    g1 = _g1(nb)
    deg_a = _deg_kernel(dst_idx[:g1], g1, nch)
    deg_b = _deg_kernel(dst_idx[g1:], nb - g1, nch)

    def views(deg, g_nb):
        return (deg.reshape(g_nb, NPAD // RB, 1, RB),
                deg[:, :n].reshape(g_nb, 5, 1, 2000))

    dv = {0: views(deg_a, g1), g1: views(deg_b, nb - g1)}

    def half(sl, g_nb, base, prev):
        deg4, deg4c = dv[base]
        y1 = _tc_first(qe_pad, deg4, W1, g_nb)
        acc1 = _aggregate_kernel(y1.reshape(g_nb * NPAD, D),
                                 src_idx[sl], dst_idx[sl], g_nb, nch)
        y2 = _tc_mid(acc1.reshape(g_nb, NPAD, D), deg4, W2, b1r, g_nb)
        acc2 = _aggregate_kernel(y2.reshape(g_nb * NPAD, D),
                                 src_idx[sl], dst_idx[sl], g_nb, nch)
        return _tc_last(acc2.reshape(g_nb, NPAD, D), deg4c, b2r,
                        g_nb, nb, base, prev)

    out = half(slice(0, g1), g1, 0, None)
    out = half(slice(g1, nb), nb - g1, g1, out)
    return out.reshape(nb * n, D)
